# baseline (speedup 1.0000x reference)
import jax
import jax.numpy as jnp
from jax.experimental import pallas as pl
from jax.experimental.pallas import tpu as pltpu

_TILE = 8192


def _mlp_kernel(xt_ref, w1t_ref, b1c_ref, w2t_ref, o_ref):
    h = jax.lax.dot_general(
        w1t_ref[...], xt_ref[...].astype(jnp.bfloat16), (((1,), (0,)), ((), ())),
        preferred_element_type=jnp.float32,
    )
    h = jnp.maximum(h + b1c_ref[...], 0.0).astype(jnp.bfloat16)
    o_ref[...] = jax.lax.dot_general(
        w2t_ref[...], h, (((1,), (0,)), ((), ())),
        preferred_element_type=jnp.float32,
    )[:4, :]


def kernel(x, w1_aug, w2_aug):
    x = jnp.asarray(x, jnp.float32)
    B = x.shape[0]
    B_pad = ((B + _TILE - 1) // _TILE) * _TILE

    xt = x.T
    if B_pad != B:
        xt = jnp.pad(xt, ((0, 0), (0, B_pad - B)))

    w1t = w1_aug[:12, :104].T.astype(jnp.bfloat16)
    b1c = w1_aug[12:13, :104].T.at[100, 0].set(1.0)
    w2t = (jnp.zeros((8, 104), jnp.float32)
           .at[:4, :100].set(w2_aug[:100, :4].T)
           .at[:4, 100].set(w2_aug[127, :4])).astype(jnp.bfloat16)

    ot = pl.pallas_call(
        _mlp_kernel,
        out_shape=jax.ShapeDtypeStruct((4, B_pad), jnp.float32),
        grid=(B_pad // _TILE,),
        in_specs=[
            pl.BlockSpec((12, _TILE), lambda i: (0, i)),
            pl.BlockSpec((104, 12), lambda i: (0, 0)),
            pl.BlockSpec((104, 1), lambda i: (0, 0)),
            pl.BlockSpec((8, 104), lambda i: (0, 0)),
        ],
        out_specs=pl.BlockSpec((4, _TILE), lambda i: (0, i)),
        compiler_params=pltpu.CompilerParams(
            dimension_semantics=("parallel",)
        ),
    )(xt, w1t, b1c, w2t)

    return ot[:, :B].T


# speedup vs baseline: 1.1783x; 1.1783x over previous
import jax
import jax.numpy as jnp
from jax.experimental import pallas as pl
from jax.experimental.pallas import tpu as pltpu

_CH = 32768
_N_STEPS = 8


def _mlp_kernel(xt_hbm, w1t_ref, b1c_ref, w2t_ref, o_hbm,
                x_buf, o_buf, in_sem, out_sem):
    base = pl.program_id(0) * (_CH * _N_STEPS)
    w1t = w1t_ref[...]
    b1c = b1c_ref[...]
    w2t = w2t_ref[...]

    def dma_in(slot, step):
        return pltpu.make_async_copy(
            xt_hbm.at[:, pl.ds(base + step * _CH, _CH)],
            x_buf.at[slot], in_sem.at[slot])

    def dma_out(slot, step):
        return pltpu.make_async_copy(
            o_buf.at[slot],
            o_hbm.at[:, pl.ds(base + step * _CH, _CH)], out_sem.at[slot])

    dma_in(0, 0).start()

    def body(step, _):
        cur = jax.lax.rem(step, 2)
        nxt = jax.lax.rem(step + 1, 2)

        @pl.when(step + 1 < _N_STEPS)
        def _():
            dma_in(nxt, step + 1).start()

        dma_in(cur, 0).wait()

        @pl.when(step >= 2)
        def _():
            dma_out(cur, 0).wait()

        xc = x_buf[cur].astype(jnp.bfloat16)
        h = jax.lax.dot_general(
            w1t, xc, (((1,), (0,)), ((), ())),
            preferred_element_type=jnp.float32,
        )
        h = jnp.maximum(h + b1c, 0.0).astype(jnp.bfloat16)
        o_buf[cur] = jax.lax.dot_general(
            w2t, h, (((1,), (0,)), ((), ())),
            preferred_element_type=jnp.float32,
        )[:4, :]

        dma_out(cur, step).start()
        return ()

    jax.lax.fori_loop(0, _N_STEPS, body, ())
    dma_out(jax.lax.rem(_N_STEPS - 2, 2), 0).wait()
    dma_out(jax.lax.rem(_N_STEPS - 1, 2), 0).wait()


def kernel(x, w1_aug, w2_aug):
    x = jnp.asarray(x, jnp.float32)
    B = x.shape[0]
    seg = 2 * _CH * _N_STEPS
    B_pad = ((B + seg - 1) // seg) * seg

    xt = x.T
    if B_pad != B:
        xt = jnp.pad(xt, ((0, 0), (0, B_pad - B)))

    w1t = w1_aug[:12, :104].T.astype(jnp.bfloat16)
    b1c = w1_aug[12:13, :104].T.at[100, 0].set(1.0)
    w2t = (jnp.zeros((8, 104), jnp.float32)
           .at[:4, :100].set(w2_aug[:100, :4].T)
           .at[:4, 100].set(w2_aug[127, :4])).astype(jnp.bfloat16)

    ot = pl.pallas_call(
        _mlp_kernel,
        out_shape=jax.ShapeDtypeStruct((4, B_pad), jnp.float32),
        grid=(2,),
        in_specs=[
            pl.BlockSpec(memory_space=pl.ANY),
            pl.BlockSpec((104, 12), lambda i: (0, 0)),
            pl.BlockSpec((104, 1), lambda i: (0, 0)),
            pl.BlockSpec((8, 104), lambda i: (0, 0)),
        ],
        out_specs=pl.BlockSpec(memory_space=pl.ANY),
        scratch_shapes=[
            pltpu.VMEM((2, 12, _CH), jnp.float32),
            pltpu.VMEM((2, 4, _CH), jnp.float32),
            pltpu.SemaphoreType.DMA((2,)),
            pltpu.SemaphoreType.DMA((2,)),
        ],
        compiler_params=pltpu.CompilerParams(
            dimension_semantics=("parallel",)
        ),
    )(xt, w1t, b1c, w2t)

    return ot[:, :B].T


# speedup vs baseline: 1.2405x; 1.0528x over previous
import jax
import jax.numpy as jnp
from jax.experimental import pallas as pl
from jax.experimental.pallas import tpu as pltpu

_TILE = 32768


def _mlp_kernel(xt_ref, w1t_ref, b1c_ref, w2t_ref, o_ref):
    h = jax.lax.dot_general(
        w1t_ref[...], xt_ref[...].astype(jnp.bfloat16), (((1,), (0,)), ((), ())),
        preferred_element_type=jnp.float32,
    )
    h = jnp.maximum(h + b1c_ref[...], 0.0).astype(jnp.bfloat16)
    o_ref[...] = jax.lax.dot_general(
        w2t_ref[...], h, (((1,), (0,)), ((), ())),
        preferred_element_type=jnp.float32,
    )[:4, :]


def kernel(x, w1_aug, w2_aug):
    x = jnp.asarray(x, jnp.float32)
    B = x.shape[0]
    B_pad = ((B + _TILE - 1) // _TILE) * _TILE

    xt = x.T
    if B_pad != B:
        xt = jnp.pad(xt, ((0, 0), (0, B_pad - B)))

    w1t = w1_aug[:12, :104].T.astype(jnp.bfloat16)
    b1c = w1_aug[12:13, :104].T.at[100, 0].set(1.0)
    w2t = (jnp.zeros((8, 104), jnp.float32)
           .at[:4, :100].set(w2_aug[:100, :4].T)
           .at[:4, 100].set(w2_aug[127, :4])).astype(jnp.bfloat16)

    ot = pl.pallas_call(
        _mlp_kernel,
        out_shape=jax.ShapeDtypeStruct((4, B_pad), jnp.float32),
        grid=(B_pad // _TILE,),
        in_specs=[
            pl.BlockSpec((12, _TILE), lambda i: (0, i)),
            pl.BlockSpec((104, 12), lambda i: (0, 0)),
            pl.BlockSpec((104, 1), lambda i: (0, 0)),
            pl.BlockSpec((8, 104), lambda i: (0, 0)),
        ],
        out_specs=pl.BlockSpec((4, _TILE), lambda i: (0, i)),
        compiler_params=pltpu.CompilerParams(
            dimension_semantics=("parallel",)
        ),
    )(xt, w1t, b1c, w2t)

    return ot[:, :B].T
